# Initial kernel scaffold; baseline (speedup 1.0000x reference)
import functools

import jax
import jax.numpy as jnp
from jax import lax
from jax.experimental import pallas as pl
from jax.experimental.pallas import tpu as pltpu
from jax.experimental.pallas import tpu_sc as plsc

N_NODES = 10000
N_EDGES = 320000
NFEAT = 128
NHID = 128
NCLASS = 40

NC = 2
NS = 16
NW = NC * NS
EPT = N_EDGES // NW
K = 100
NCH = EPT // K
ROWS_PER_TILE = N_NODES // NS
ZROWS = 125
LANES = 16


def _sc_degree(dst2):
    mesh = plsc.VectorSubcoreMesh(core_axis_name="c", subcore_axis_name="s")

    @functools.partial(
        pl.kernel,
        out_type=jax.ShapeDtypeStruct((NW, N_NODES), jnp.float32),
        mesh=mesh,
        scratch_types=[
            pltpu.VMEM((EPT,), jnp.int32),
            pltpu.VMEM((N_NODES,), jnp.float32),
        ],
    )
    def deg_kernel(dst_hbm, out_hbm, idx_v, deg_v):
        c = lax.axis_index("c")
        s = lax.axis_index("s")
        wid = c * NS + s
        pltpu.sync_copy(dst_hbm.at[wid], idx_v)

        def zero_body(i, carry):
            deg_v[pl.ds(i * LANES, LANES)] = jnp.zeros((LANES,), jnp.float32)
            return carry

        lax.fori_loop(0, N_NODES // LANES, zero_body, 0)

        ones = jnp.ones((LANES,), jnp.float32)

        def acc_body(i, carry):
            idx = idx_v[pl.ds(i * LANES, LANES)]
            plsc.addupdate_scatter(deg_v, [idx], ones)
            return carry

        lax.fori_loop(0, EPT // LANES, acc_body, 0)
        pltpu.sync_copy(deg_v, out_hbm.at[wid])

    return deg_kernel(dst2)


def _sc_aggregate(y, src3, dst3):
    mesh = plsc.VectorSubcoreMesh(core_axis_name="c", subcore_axis_name="s")

    @functools.partial(
        pl.kernel,
        out_type=jax.ShapeDtypeStruct((NC, N_NODES, NHID), jnp.float32),
        mesh=mesh,
        scratch_types=[
            pltpu.VMEM((NCH, K), jnp.int32),
            pltpu.VMEM((NCH, K), jnp.int32),
            pltpu.VMEM((K, NHID), jnp.float32),
            pltpu.VMEM((ZROWS, NHID), jnp.float32),
            pltpu.VMEM_SHARED((N_NODES, NHID), jnp.float32),
            pltpu.SemaphoreType.DMA,
        ],
    )
    def agg_kernel(y_hbm, src_hbm, dst_hbm, out_hbm,
                   src_v, dst_v, buf, zbuf, accum, sem):
        c = lax.axis_index("c")
        s = lax.axis_index("s")
        wid = c * NS + s
        pltpu.sync_copy(src_hbm.at[wid], src_v)
        pltpu.sync_copy(dst_hbm.at[wid], dst_v)

        def zb(i, carry):
            r = i // (NHID // LANES)
            q = i % (NHID // LANES)
            zbuf[r, pl.ds(q * LANES, LANES)] = jnp.zeros((LANES,), jnp.float32)
            return carry

        lax.fori_loop(0, ZROWS * (NHID // LANES), zb, 0)
        base = s * ROWS_PER_TILE
        for j in range(ROWS_PER_TILE // ZROWS):
            pltpu.sync_copy(zbuf, accum.at[pl.ds(base + j * ZROWS, ZROWS)])
        plsc.subcore_barrier()

        def step(i, carry):
            pltpu.async_copy(y_hbm.at[src_v.at[i]], buf, sem).wait()
            pltpu.sync_copy(buf, accum.at[dst_v.at[i]], add=True)
            return carry

        lax.fori_loop(0, NCH, step, 0)
        plsc.subcore_barrier()

        pltpu.sync_copy(accum.at[pl.ds(base, ROWS_PER_TILE)],
                        out_hbm.at[c, pl.ds(base, ROWS_PER_TILE)])

    return agg_kernel(y, src3, dst3)


_ROWS = 2000


def _tc_prep(deg_part, x, W1):
    def body(degp_ref, x_ref, w_ref, dinv_ref, y_ref):
        deg = jnp.sum(degp_ref[...], axis=0) + 1.0
        dinv = lax.rsqrt(deg)
        dinv_ref[...] = dinv[:, None]
        xw = jnp.dot(x_ref[...], w_ref[...], preferred_element_type=jnp.float32)
        y_ref[...] = xw * dinv[:, None]

    return pl.pallas_call(
        body,
        grid=(N_NODES // _ROWS,),
        in_specs=[
            pl.BlockSpec((NW, _ROWS), lambda i: (0, i)),
            pl.BlockSpec((_ROWS, NFEAT), lambda i: (i, 0)),
            pl.BlockSpec((NFEAT, NHID), lambda i: (0, 0)),
        ],
        out_specs=[
            pl.BlockSpec((_ROWS, 1), lambda i: (i, 0)),
            pl.BlockSpec((_ROWS, NHID), lambda i: (i, 0)),
        ],
        out_shape=[
            jax.ShapeDtypeStruct((N_NODES, 1), jnp.float32),
            jax.ShapeDtypeStruct((N_NODES, NHID), jnp.float32),
        ],
    )(deg_part, x, W1)


def _tc_mid(part, y_prev, dinv, b, Wn):
    def body(p_ref, y_ref, dinv_ref, b_ref, w_ref, out_ref):
        agg = p_ref[0] + p_ref[1] + y_ref[...]
        h = jnp.maximum(agg * dinv_ref[...] + b_ref[...], 0.0)
        out_ref[...] = jnp.dot(
            h, w_ref[...], preferred_element_type=jnp.float32) * dinv_ref[...]

    return pl.pallas_call(
        body,
        grid=(N_NODES // _ROWS,),
        in_specs=[
            pl.BlockSpec((NC, _ROWS, NHID), lambda i: (0, i, 0)),
            pl.BlockSpec((_ROWS, NHID), lambda i: (i, 0)),
            pl.BlockSpec((_ROWS, 1), lambda i: (i, 0)),
            pl.BlockSpec((1, NHID), lambda i: (0, 0)),
            pl.BlockSpec((NHID, NHID), lambda i: (0, 0)),
        ],
        out_specs=pl.BlockSpec((_ROWS, NHID), lambda i: (i, 0)),
        out_shape=jax.ShapeDtypeStruct((N_NODES, NHID), jnp.float32),
    )(part, y_prev, dinv, b, Wn)


def _tc_final(part, y_prev, dinv, b2, Wl, bl):
    def body(p_ref, y_ref, dinv_ref, b_ref, wl_ref, bl_ref, out_ref):
        agg = p_ref[0] + p_ref[1] + y_ref[...]
        h = jnp.maximum(agg * dinv_ref[...] + b_ref[...], 0.0)
        logits = jnp.dot(h, wl_ref[...],
                         preferred_element_type=jnp.float32) + bl_ref[...]
        m = jnp.max(logits, axis=1, keepdims=True)
        e = jnp.exp(logits - m)
        out_ref[...] = e / jnp.sum(e, axis=1, keepdims=True)

    return pl.pallas_call(
        body,
        grid=(N_NODES // _ROWS,),
        in_specs=[
            pl.BlockSpec((NC, _ROWS, NHID), lambda i: (0, i, 0)),
            pl.BlockSpec((_ROWS, NHID), lambda i: (i, 0)),
            pl.BlockSpec((_ROWS, 1), lambda i: (i, 0)),
            pl.BlockSpec((1, NHID), lambda i: (0, 0)),
            pl.BlockSpec((NHID, NCLASS), lambda i: (0, 0)),
            pl.BlockSpec((1, NCLASS), lambda i: (0, 0)),
        ],
        out_specs=pl.BlockSpec((_ROWS, NCLASS), lambda i: (i, 0)),
        out_shape=jax.ShapeDtypeStruct((N_NODES, NCLASS), jnp.float32),
    )(part, y_prev, dinv, b2, Wl, bl)


def kernel(x, edge_index, W1, b1, W2, b2, Wl, bl):
    src = edge_index[0].astype(jnp.int32)
    dst = edge_index[1].astype(jnp.int32)
    dst2 = dst.reshape(NW, EPT)
    src3 = src.reshape(NW, NCH, K)
    dst3 = dst.reshape(NW, NCH, K)

    deg_part = _sc_degree(dst2)
    dinv, y1 = _tc_prep(deg_part, x, W1)
    part1 = _sc_aggregate(y1, src3, dst3)
    y2 = _tc_mid(part1, y1, dinv, b1.reshape(1, NHID), W2)
    part2 = _sc_aggregate(y2, src3, dst3)
    return _tc_final(part2, y2, dinv, b2.reshape(1, NHID),
                     Wl, bl.reshape(1, NCLASS))


# speedup vs baseline: 20.1303x; 20.1303x over previous
import functools

import jax
import jax.numpy as jnp
from jax import lax
from jax.experimental import pallas as pl
from jax.experimental.pallas import tpu as pltpu
from jax.experimental.pallas import tpu_sc as plsc

N_NODES = 10000
N_EDGES = 320000
NFEAT = 128
NHID = 128
NCLASS = 40

NC = 2
NS = 16
NW = NC * NS
EPT = N_EDGES // NW
K = 80
NCH = EPT // K
N_PAD = 10240
ROWS_PER_TILE = N_PAD // NS
LANES = 16


def _sc_degree(dst2):
    mesh = plsc.VectorSubcoreMesh(core_axis_name="c", subcore_axis_name="s")

    @functools.partial(
        pl.kernel,
        out_type=jax.ShapeDtypeStruct((NW, N_NODES), jnp.float32),
        mesh=mesh,
        scratch_types=[
            pltpu.VMEM((EPT,), jnp.int32),
            pltpu.VMEM((N_NODES,), jnp.float32),
        ],
        compiler_params=pltpu.CompilerParams(needs_layout_passes=False),
    )
    def deg_kernel(dst_hbm, out_hbm, idx_v, deg_v):
        c = lax.axis_index("c")
        s = lax.axis_index("s")
        wid = c * NS + s
        pltpu.sync_copy(dst_hbm.at[wid], idx_v)

        def zero_body(i, carry):
            deg_v[pl.ds(i * LANES, LANES)] = jnp.zeros((LANES,), jnp.float32)
            return carry

        lax.fori_loop(0, N_NODES // LANES, zero_body, 0)

        ones = jnp.ones((LANES,), jnp.float32)

        def acc_body(i, carry):
            idx = idx_v[pl.ds(i * LANES, LANES)]
            plsc.addupdate_scatter(deg_v, [idx], ones)
            return carry

        lax.fori_loop(0, EPT // LANES, acc_body, 0)
        pltpu.sync_copy(deg_v, out_hbm.at[wid])

    return deg_kernel(dst2)


def _sc_aggregate(y, src3, dst3):
    mesh = plsc.VectorSubcoreMesh(core_axis_name="c", subcore_axis_name="s")

    @functools.partial(
        pl.kernel,
        out_type=jax.ShapeDtypeStruct((NC, N_PAD, NHID), jnp.float32),
        mesh=mesh,
        scratch_types=[
            pltpu.VMEM((NCH, K), jnp.int32),
            pltpu.VMEM((NCH, K), jnp.int32),
            pltpu.VMEM((K, NHID), jnp.float32),
            pltpu.VMEM_SHARED((N_PAD, NHID), jnp.float32),
            pltpu.SemaphoreType.DMA,
        ],
    )
    def agg_kernel(y_hbm, src_hbm, dst_hbm, out_hbm,
                   src_v, dst_v, buf, accum, sem):
        c = lax.axis_index("c")
        s = lax.axis_index("s")
        wid = c * NS + s
        pltpu.sync_copy(src_hbm.at[wid], src_v)
        pltpu.sync_copy(dst_hbm.at[wid], dst_v)

        def zb(i, carry):
            r = i // (NHID // LANES)
            q = i % (NHID // LANES)
            buf[r, pl.ds(q * LANES, LANES)] = jnp.zeros((LANES,), jnp.float32)
            return carry

        lax.fori_loop(0, K * (NHID // LANES), zb, 0)
        base = s * ROWS_PER_TILE
        for j in range(ROWS_PER_TILE // K):
            pltpu.sync_copy(buf, accum.at[pl.ds(base + j * K, K)])
        plsc.subcore_barrier()

        def step(i, carry):
            pltpu.async_copy(y_hbm.at[src_v.at[i]], buf, sem).wait()
            pltpu.sync_copy(buf, accum.at[dst_v.at[i]], add=True)
            return carry

        lax.fori_loop(0, NCH, step, 0)
        plsc.subcore_barrier()

        pltpu.sync_copy(accum.at[pl.ds(base, ROWS_PER_TILE)],
                        out_hbm.at[c, pl.ds(base, ROWS_PER_TILE)])

    return agg_kernel(y, src3, dst3)


_ROWS = 2000


def _tc_prep(deg_part_t, x, W1):
    def body(degp_ref, x_ref, w_ref, dinv_ref, y_ref):
        deg = jnp.sum(degp_ref[...], axis=1) + 1.0
        dinv = lax.rsqrt(deg)
        dinv_ref[...] = dinv[:, None]
        xw = jnp.dot(x_ref[...], w_ref[...], preferred_element_type=jnp.float32)
        y_ref[...] = xw * dinv[:, None]

    return pl.pallas_call(
        body,
        grid=(N_NODES // _ROWS,),
        in_specs=[
            pl.BlockSpec((_ROWS, NW), lambda i: (i, 0)),
            pl.BlockSpec((_ROWS, NFEAT), lambda i: (i, 0)),
            pl.BlockSpec((NFEAT, NHID), lambda i: (0, 0)),
        ],
        out_specs=[
            pl.BlockSpec((_ROWS, 1), lambda i: (i, 0)),
            pl.BlockSpec((_ROWS, NHID), lambda i: (i, 0)),
        ],
        out_shape=[
            jax.ShapeDtypeStruct((N_NODES, 1), jnp.float32),
            jax.ShapeDtypeStruct((N_NODES, NHID), jnp.float32),
        ],
    )(deg_part_t, x, W1)


def _tc_mid(part, y_prev, dinv, b, Wn):
    def body(p_ref, y_ref, dinv_ref, b_ref, w_ref, out_ref):
        agg = p_ref[0] + p_ref[1] + y_ref[...]
        h = jnp.maximum(agg * dinv_ref[...] + b_ref[...], 0.0)
        out_ref[...] = jnp.dot(
            h, w_ref[...], preferred_element_type=jnp.float32) * dinv_ref[...]

    return pl.pallas_call(
        body,
        grid=(N_NODES // _ROWS,),
        in_specs=[
            pl.BlockSpec((NC, _ROWS, NHID), lambda i: (0, i, 0)),
            pl.BlockSpec((_ROWS, NHID), lambda i: (i, 0)),
            pl.BlockSpec((_ROWS, 1), lambda i: (i, 0)),
            pl.BlockSpec((1, NHID), lambda i: (0, 0)),
            pl.BlockSpec((NHID, NHID), lambda i: (0, 0)),
        ],
        out_specs=pl.BlockSpec((_ROWS, NHID), lambda i: (i, 0)),
        out_shape=jax.ShapeDtypeStruct((N_NODES, NHID), jnp.float32),
    )(part, y_prev, dinv, b, Wn)


def _tc_final(part, y_prev, dinv, b2, Wl, bl):
    def body(p_ref, y_ref, dinv_ref, b_ref, wl_ref, bl_ref, out_ref):
        agg = p_ref[0] + p_ref[1] + y_ref[...]
        h = jnp.maximum(agg * dinv_ref[...] + b_ref[...], 0.0)
        logits = jnp.dot(h, wl_ref[...],
                         preferred_element_type=jnp.float32) + bl_ref[...]
        m = jnp.max(logits, axis=1, keepdims=True)
        e = jnp.exp(logits - m)
        out_ref[...] = e / jnp.sum(e, axis=1, keepdims=True)

    return pl.pallas_call(
        body,
        grid=(N_NODES // _ROWS,),
        in_specs=[
            pl.BlockSpec((NC, _ROWS, NHID), lambda i: (0, i, 0)),
            pl.BlockSpec((_ROWS, NHID), lambda i: (i, 0)),
            pl.BlockSpec((_ROWS, 1), lambda i: (i, 0)),
            pl.BlockSpec((1, NHID), lambda i: (0, 0)),
            pl.BlockSpec((NHID, NCLASS), lambda i: (0, 0)),
            pl.BlockSpec((1, NCLASS), lambda i: (0, 0)),
        ],
        out_specs=pl.BlockSpec((_ROWS, NCLASS), lambda i: (i, 0)),
        out_shape=jax.ShapeDtypeStruct((N_NODES, NCLASS), jnp.float32),
    )(part, y_prev, dinv, b2, Wl, bl)


def kernel(x, edge_index, W1, b1, W2, b2, Wl, bl):
    src = edge_index[0].astype(jnp.int32)
    dst = edge_index[1].astype(jnp.int32)
    dst2 = dst.reshape(NW, EPT)
    src3 = src.reshape(NW, NCH, K)
    dst3 = dst.reshape(NW, NCH, K)

    deg_part = _sc_degree(dst2)
    dinv, y1 = _tc_prep(deg_part.T, x, W1)
    part1 = _sc_aggregate(y1, src3, dst3)
    y2 = _tc_mid(part1, y1, dinv, b1.reshape(1, NHID), W2)
    part2 = _sc_aggregate(y2, src3, dst3)
    return _tc_final(part2, y2, dinv, b2.reshape(1, NHID),
                     Wl, bl.reshape(1, NCLASS))


# speedup vs baseline: 31.3673x; 1.5582x over previous
import functools

import jax
import jax.numpy as jnp
from jax import lax
from jax.experimental import pallas as pl
from jax.experimental.pallas import tpu as pltpu
from jax.experimental.pallas import tpu_sc as plsc

N_NODES = 10000
N_EDGES = 320000
NFEAT = 128
NHID = 128
NCLASS = 40

NC = 2
NS = 16
NW = NC * NS
EPT = N_EDGES // NW
K = 80
NCH = EPT // K
N_PAD = 10240
ROWS_PER_TILE = N_PAD // NS
LANES = 16


def _sc_degree(dst2):
    mesh = plsc.VectorSubcoreMesh(core_axis_name="c", subcore_axis_name="s")

    @functools.partial(
        pl.kernel,
        out_type=jax.ShapeDtypeStruct((NW, N_NODES), jnp.float32),
        mesh=mesh,
        scratch_types=[
            pltpu.VMEM((EPT,), jnp.int32),
            pltpu.VMEM((N_NODES,), jnp.float32),
        ],
        compiler_params=pltpu.CompilerParams(needs_layout_passes=False),
    )
    def deg_kernel(dst_hbm, out_hbm, idx_v, deg_v):
        c = lax.axis_index("c")
        s = lax.axis_index("s")
        wid = c * NS + s
        pltpu.sync_copy(dst_hbm.at[wid], idx_v)

        def zero_body(i, carry):
            deg_v[pl.ds(i * LANES, LANES)] = jnp.zeros((LANES,), jnp.float32)
            return carry

        lax.fori_loop(0, N_NODES // LANES, zero_body, 0)

        ones = jnp.ones((LANES,), jnp.float32)

        def acc_body(i, carry):
            idx = idx_v[pl.ds(i * LANES, LANES)]
            plsc.addupdate_scatter(deg_v, [idx], ones)
            return carry

        lax.fori_loop(0, EPT // LANES, acc_body, 0)
        pltpu.sync_copy(deg_v, out_hbm.at[wid])

    return deg_kernel(dst2)


def _sc_aggregate(y, src3, dst4):
    mesh = plsc.VectorSubcoreMesh(core_axis_name="c", subcore_axis_name="s")

    @functools.partial(
        pl.kernel,
        out_type=jax.ShapeDtypeStruct((NC, N_PAD, NHID), jnp.float32),
        mesh=mesh,
        scratch_types=[
            pltpu.VMEM((NCH, K), jnp.int32),
            pltpu.VMEM((1, K), jnp.int32),
            pltpu.VMEM((1, K), jnp.int32),
            pltpu.VMEM((K, NHID), jnp.float32),
            pltpu.VMEM((K, NHID), jnp.float32),
            pltpu.VMEM_SHARED((N_PAD, NHID), jnp.float32),
            pltpu.SemaphoreType.DMA,
            pltpu.SemaphoreType.DMA,
            pltpu.SemaphoreType.DMA,
            pltpu.SemaphoreType.DMA,
        ],
    )
    def agg_kernel(y_hbm, src_hbm, dst_hbm, out_hbm,
                   src_v, db0, db1, buf0, buf1, accum,
                   gsem0, gsem1, dsem0, dsem1):
        c = lax.axis_index("c")
        s = lax.axis_index("s")
        wid = c * NS + s
        pltpu.sync_copy(src_hbm.at[wid], src_v)

        def zb(i, carry):
            r = i // (NHID // LANES)
            q = i % (NHID // LANES)
            buf0[r, pl.ds(q * LANES, LANES)] = jnp.zeros((LANES,), jnp.float32)
            return carry

        lax.fori_loop(0, K * (NHID // LANES), zb, 0)
        base = s * ROWS_PER_TILE

        def zcopy(j, carry):
            off = pl.multiple_of(base + j * K, 8)
            pltpu.sync_copy(buf0, accum.at[pl.ds(off, K)])
            return carry

        lax.fori_loop(0, ROWS_PER_TILE // K, zcopy, 0)
        plsc.subcore_barrier()

        pltpu.async_copy(dst_hbm.at[wid, 0], db0, dsem0)
        pltpu.async_copy(y_hbm.at[src_v.at[0]], buf0, gsem0)

        def pair(i2, carry):
            i = 2 * i2
            pltpu.async_copy(dst_hbm.at[wid, i + 1], db1, dsem1)
            pltpu.async_copy(y_hbm.at[src_v.at[i + 1]], buf1, gsem1)
            pltpu.make_async_copy(y_hbm.at[src_v.at[i]], buf0, gsem0).wait()
            pltpu.make_async_copy(dst_hbm.at[wid, i], db0, dsem0).wait()
            pltpu.sync_copy(buf0, accum.at[db0.at[0]], add=True)
            pltpu.async_copy(dst_hbm.at[wid, i + 2], db0, dsem0)
            pltpu.async_copy(y_hbm.at[src_v.at[i + 2]], buf0, gsem0)
            pltpu.make_async_copy(y_hbm.at[src_v.at[i + 1]], buf1, gsem1).wait()
            pltpu.make_async_copy(dst_hbm.at[wid, i + 1], db1, dsem1).wait()
            pltpu.sync_copy(buf1, accum.at[db1.at[0]], add=True)
            return carry

        lax.fori_loop(0, NCH // 2, pair, 0)
        pltpu.make_async_copy(y_hbm.at[src_v.at[NCH - 1]], buf0, gsem0).wait()
        pltpu.make_async_copy(dst_hbm.at[wid, NCH - 1], db0, dsem0).wait()
        pltpu.sync_copy(buf0, accum.at[db0.at[0]], add=True)
        plsc.subcore_barrier()

        pltpu.sync_copy(accum.at[pl.ds(base, ROWS_PER_TILE)],
                        out_hbm.at[c, pl.ds(base, ROWS_PER_TILE)])

    return agg_kernel(y, src3, dst4)


_ROWS = 2000


def _tc_prep(deg_part_t, x, W1):
    def body(degp_ref, x_ref, w_ref, dinv_ref, y_ref):
        deg = jnp.sum(degp_ref[...], axis=1) + 1.0
        dinv = lax.rsqrt(deg)
        dinv_ref[...] = dinv[:, None]
        xw = jnp.dot(x_ref[...], w_ref[...], preferred_element_type=jnp.float32)
        y_ref[...] = xw * dinv[:, None]

    return pl.pallas_call(
        body,
        grid=(N_NODES // _ROWS,),
        in_specs=[
            pl.BlockSpec((_ROWS, NW), lambda i: (i, 0)),
            pl.BlockSpec((_ROWS, NFEAT), lambda i: (i, 0)),
            pl.BlockSpec((NFEAT, NHID), lambda i: (0, 0)),
        ],
        out_specs=[
            pl.BlockSpec((_ROWS, 1), lambda i: (i, 0)),
            pl.BlockSpec((_ROWS, NHID), lambda i: (i, 0)),
        ],
        out_shape=[
            jax.ShapeDtypeStruct((N_NODES, 1), jnp.float32),
            jax.ShapeDtypeStruct((N_NODES, NHID), jnp.float32),
        ],
    )(deg_part_t, x, W1)


def _tc_mid(part, y_prev, dinv, b, Wn):
    def body(p_ref, y_ref, dinv_ref, b_ref, w_ref, out_ref):
        agg = p_ref[0] + p_ref[1] + y_ref[...]
        h = jnp.maximum(agg * dinv_ref[...] + b_ref[...], 0.0)
        out_ref[...] = jnp.dot(
            h, w_ref[...], preferred_element_type=jnp.float32) * dinv_ref[...]

    return pl.pallas_call(
        body,
        grid=(N_NODES // _ROWS,),
        in_specs=[
            pl.BlockSpec((NC, _ROWS, NHID), lambda i: (0, i, 0)),
            pl.BlockSpec((_ROWS, NHID), lambda i: (i, 0)),
            pl.BlockSpec((_ROWS, 1), lambda i: (i, 0)),
            pl.BlockSpec((1, NHID), lambda i: (0, 0)),
            pl.BlockSpec((NHID, NHID), lambda i: (0, 0)),
        ],
        out_specs=pl.BlockSpec((_ROWS, NHID), lambda i: (i, 0)),
        out_shape=jax.ShapeDtypeStruct((N_NODES, NHID), jnp.float32),
    )(part, y_prev, dinv, b, Wn)


def _tc_final(part, y_prev, dinv, b2, Wl, bl):
    def body(p_ref, y_ref, dinv_ref, b_ref, wl_ref, bl_ref, out_ref):
        agg = p_ref[0] + p_ref[1] + y_ref[...]
        h = jnp.maximum(agg * dinv_ref[...] + b_ref[...], 0.0)
        logits = jnp.dot(h, wl_ref[...],
                         preferred_element_type=jnp.float32) + bl_ref[...]
        m = jnp.max(logits, axis=1, keepdims=True)
        e = jnp.exp(logits - m)
        out_ref[...] = e / jnp.sum(e, axis=1, keepdims=True)

    return pl.pallas_call(
        body,
        grid=(N_NODES // _ROWS,),
        in_specs=[
            pl.BlockSpec((NC, _ROWS, NHID), lambda i: (0, i, 0)),
            pl.BlockSpec((_ROWS, NHID), lambda i: (i, 0)),
            pl.BlockSpec((_ROWS, 1), lambda i: (i, 0)),
            pl.BlockSpec((1, NHID), lambda i: (0, 0)),
            pl.BlockSpec((NHID, NCLASS), lambda i: (0, 0)),
            pl.BlockSpec((1, NCLASS), lambda i: (0, 0)),
        ],
        out_specs=pl.BlockSpec((_ROWS, NCLASS), lambda i: (i, 0)),
        out_shape=jax.ShapeDtypeStruct((N_NODES, NCLASS), jnp.float32),
    )(part, y_prev, dinv, b2, Wl, bl)


def kernel(x, edge_index, W1, b1, W2, b2, Wl, bl):
    src = edge_index[0].astype(jnp.int32)
    dst = edge_index[1].astype(jnp.int32)
    dst2 = dst.reshape(NW, EPT)
    src3 = src.reshape(NW, NCH, K)
    dst4 = dst.reshape(NW, NCH, 1, K)

    deg_part = _sc_degree(dst2)
    dinv, y1 = _tc_prep(deg_part.T, x, W1)
    part1 = _sc_aggregate(y1, src3, dst4)
    y2 = _tc_mid(part1, y1, dinv, b1.reshape(1, NHID), W2)
    part2 = _sc_aggregate(y2, src3, dst4)
    return _tc_final(part2, y2, dinv, b2.reshape(1, NHID),
                     Wl, bl.reshape(1, NCLASS))


# speedup vs baseline: 32.8256x; 1.0465x over previous
import functools

import jax
import jax.numpy as jnp
from jax import lax
from jax.experimental import pallas as pl
from jax.experimental.pallas import tpu as pltpu
from jax.experimental.pallas import tpu_sc as plsc

N_NODES = 10000
N_EDGES = 320000
NFEAT = 128
NHID = 128
NCLASS = 40

NC = 2
NS = 16
NW = NC * NS
EPT = N_EDGES // NW
K = 100
NCH = EPT // K
N_PAD = 10240
ROWS_PER_TILE = N_PAD // NS
ZROWS = 80
LANES = 16


def _sc_degree(dst2):
    mesh = plsc.VectorSubcoreMesh(core_axis_name="c", subcore_axis_name="s")

    @functools.partial(
        pl.kernel,
        out_type=jax.ShapeDtypeStruct((NW, N_NODES), jnp.float32),
        mesh=mesh,
        scratch_types=[
            pltpu.VMEM((EPT,), jnp.int32),
            pltpu.VMEM((N_NODES,), jnp.float32),
        ],
        compiler_params=pltpu.CompilerParams(needs_layout_passes=False),
    )
    def deg_kernel(dst_hbm, out_hbm, idx_v, deg_v):
        c = lax.axis_index("c")
        s = lax.axis_index("s")
        wid = c * NS + s
        pltpu.sync_copy(dst_hbm.at[wid], idx_v)

        def zero_body(i, carry):
            deg_v[pl.ds(i * LANES, LANES)] = jnp.zeros((LANES,), jnp.float32)
            return carry

        lax.fori_loop(0, N_NODES // LANES, zero_body, 0)

        ones = jnp.ones((LANES,), jnp.float32)

        def acc_body(i, carry):
            idx = idx_v[pl.ds(i * LANES, LANES)]
            plsc.addupdate_scatter(deg_v, [idx], ones)
            return carry

        lax.fori_loop(0, EPT // LANES, acc_body, 0)
        pltpu.sync_copy(deg_v, out_hbm.at[wid])

    return deg_kernel(dst2)


def _sc_aggregate(y, src3, dst4):
    mesh = plsc.VectorSubcoreMesh(core_axis_name="c", subcore_axis_name="s")

    @functools.partial(
        pl.kernel,
        out_type=jax.ShapeDtypeStruct((NC, N_PAD, NHID), jnp.float32),
        mesh=mesh,
        scratch_types=[
            pltpu.VMEM((NCH, K), jnp.int32),
            pltpu.VMEM((1, K), jnp.int32),
            pltpu.VMEM((1, K), jnp.int32),
            pltpu.VMEM((K, NHID), jnp.float32),
            pltpu.VMEM((K, NHID), jnp.float32),
            pltpu.VMEM_SHARED((N_PAD, NHID), jnp.float32),
            pltpu.SemaphoreType.DMA,
            pltpu.SemaphoreType.DMA,
            pltpu.SemaphoreType.DMA,
            pltpu.SemaphoreType.DMA,
        ],
    )
    def agg_kernel(y_hbm, src_hbm, dst_hbm, out_hbm,
                   src_v, db0, db1, buf0, buf1, accum,
                   gsem0, gsem1, dsem0, dsem1):
        c = lax.axis_index("c")
        s = lax.axis_index("s")
        wid = c * NS + s
        pltpu.sync_copy(src_hbm.at[wid], src_v)

        def zb(i, carry):
            r = i // (NHID // LANES)
            q = i % (NHID // LANES)
            buf0[r, pl.ds(q * LANES, LANES)] = jnp.zeros((LANES,), jnp.float32)
            return carry

        lax.fori_loop(0, ZROWS * (NHID // LANES), zb, 0)
        base = s * ROWS_PER_TILE

        def zcopy(j, carry):
            off = pl.multiple_of(base + j * ZROWS, 8)
            pltpu.sync_copy(buf0.at[pl.ds(0, ZROWS)], accum.at[pl.ds(off, ZROWS)])
            return carry

        lax.fori_loop(0, ROWS_PER_TILE // ZROWS, zcopy, 0)
        plsc.subcore_barrier()

        pltpu.async_copy(dst_hbm.at[wid, 0], db0, dsem0)
        pltpu.async_copy(y_hbm.at[src_v.at[0]], buf0, gsem0)

        def pair(i2, carry):
            i = 2 * i2
            pltpu.async_copy(dst_hbm.at[wid, i + 1], db1, dsem1)
            pltpu.async_copy(y_hbm.at[src_v.at[i + 1]], buf1, gsem1)
            pltpu.make_async_copy(y_hbm.at[src_v.at[i]], buf0, gsem0).wait()
            pltpu.make_async_copy(dst_hbm.at[wid, i], db0, dsem0).wait()
            pltpu.sync_copy(buf0, accum.at[db0.at[0]], add=True)

            @pl.when(i2 + 1 < NCH // 2)
            def _():
                pltpu.async_copy(dst_hbm.at[wid, i + 2], db0, dsem0)
                pltpu.async_copy(y_hbm.at[src_v.at[i + 2]], buf0, gsem0)

            pltpu.make_async_copy(y_hbm.at[src_v.at[i + 1]], buf1, gsem1).wait()
            pltpu.make_async_copy(dst_hbm.at[wid, i + 1], db1, dsem1).wait()
            pltpu.sync_copy(buf1, accum.at[db1.at[0]], add=True)
            return carry

        lax.fori_loop(0, NCH // 2, pair, 0)
        plsc.subcore_barrier()

        pltpu.sync_copy(accum.at[pl.ds(base, ROWS_PER_TILE)],
                        out_hbm.at[c, pl.ds(base, ROWS_PER_TILE)])

    return agg_kernel(y, src3, dst4)


_ROWS = 2000


def _tc_prep(deg_part_t, x, W1):
    def body(degp_ref, x_ref, w_ref, dinv_ref, y_ref):
        deg = jnp.sum(degp_ref[...], axis=1) + 1.0
        dinv = lax.rsqrt(deg)
        dinv_ref[...] = dinv[:, None]
        xw = jnp.dot(x_ref[...], w_ref[...], preferred_element_type=jnp.float32)
        y_ref[...] = xw * dinv[:, None]

    return pl.pallas_call(
        body,
        grid=(N_NODES // _ROWS,),
        in_specs=[
            pl.BlockSpec((_ROWS, NW), lambda i: (i, 0)),
            pl.BlockSpec((_ROWS, NFEAT), lambda i: (i, 0)),
            pl.BlockSpec((NFEAT, NHID), lambda i: (0, 0)),
        ],
        out_specs=[
            pl.BlockSpec((_ROWS, 1), lambda i: (i, 0)),
            pl.BlockSpec((_ROWS, NHID), lambda i: (i, 0)),
        ],
        out_shape=[
            jax.ShapeDtypeStruct((N_NODES, 1), jnp.float32),
            jax.ShapeDtypeStruct((N_NODES, NHID), jnp.float32),
        ],
    )(deg_part_t, x, W1)


def _tc_mid(part, y_prev, dinv, b, Wn):
    def body(p_ref, y_ref, dinv_ref, b_ref, w_ref, out_ref):
        agg = p_ref[0] + p_ref[1] + y_ref[...]
        h = jnp.maximum(agg * dinv_ref[...] + b_ref[...], 0.0)
        out_ref[...] = jnp.dot(
            h, w_ref[...], preferred_element_type=jnp.float32) * dinv_ref[...]

    return pl.pallas_call(
        body,
        grid=(N_NODES // _ROWS,),
        in_specs=[
            pl.BlockSpec((NC, _ROWS, NHID), lambda i: (0, i, 0)),
            pl.BlockSpec((_ROWS, NHID), lambda i: (i, 0)),
            pl.BlockSpec((_ROWS, 1), lambda i: (i, 0)),
            pl.BlockSpec((1, NHID), lambda i: (0, 0)),
            pl.BlockSpec((NHID, NHID), lambda i: (0, 0)),
        ],
        out_specs=pl.BlockSpec((_ROWS, NHID), lambda i: (i, 0)),
        out_shape=jax.ShapeDtypeStruct((N_NODES, NHID), jnp.float32),
    )(part, y_prev, dinv, b, Wn)


def _tc_final(part, y_prev, dinv, b2, Wl, bl):
    def body(p_ref, y_ref, dinv_ref, b_ref, wl_ref, bl_ref, out_ref):
        agg = p_ref[0] + p_ref[1] + y_ref[...]
        h = jnp.maximum(agg * dinv_ref[...] + b_ref[...], 0.0)
        logits = jnp.dot(h, wl_ref[...],
                         preferred_element_type=jnp.float32) + bl_ref[...]
        m = jnp.max(logits, axis=1, keepdims=True)
        e = jnp.exp(logits - m)
        out_ref[...] = e / jnp.sum(e, axis=1, keepdims=True)

    return pl.pallas_call(
        body,
        grid=(N_NODES // _ROWS,),
        in_specs=[
            pl.BlockSpec((NC, _ROWS, NHID), lambda i: (0, i, 0)),
            pl.BlockSpec((_ROWS, NHID), lambda i: (i, 0)),
            pl.BlockSpec((_ROWS, 1), lambda i: (i, 0)),
            pl.BlockSpec((1, NHID), lambda i: (0, 0)),
            pl.BlockSpec((NHID, NCLASS), lambda i: (0, 0)),
            pl.BlockSpec((1, NCLASS), lambda i: (0, 0)),
        ],
        out_specs=pl.BlockSpec((_ROWS, NCLASS), lambda i: (i, 0)),
        out_shape=jax.ShapeDtypeStruct((N_NODES, NCLASS), jnp.float32),
    )(part, y_prev, dinv, b2, Wl, bl)


def kernel(x, edge_index, W1, b1, W2, b2, Wl, bl):
    src = edge_index[0].astype(jnp.int32)
    dst = edge_index[1].astype(jnp.int32)
    dst2 = dst.reshape(NW, EPT)
    src3 = src.reshape(NW, NCH, K)
    dst4 = dst.reshape(NW, NCH, 1, K)

    deg_part = _sc_degree(dst2)
    dinv, y1 = _tc_prep(deg_part.T, x, W1)
    part1 = _sc_aggregate(y1, src3, dst4)
    y2 = _tc_mid(part1, y1, dinv, b1.reshape(1, NHID), W2)
    part2 = _sc_aggregate(y2, src3, dst4)
    return _tc_final(part2, y2, dinv, b2.reshape(1, NHID),
                     Wl, bl.reshape(1, NCLASS))


# speedup vs baseline: 33.3613x; 1.0163x over previous
import functools

import jax
import jax.numpy as jnp
from jax import lax
from jax.experimental import pallas as pl
from jax.experimental.pallas import tpu as pltpu
from jax.experimental.pallas import tpu_sc as plsc

N_NODES = 10000
N_EDGES = 320000
NFEAT = 128
NHID = 128
NCLASS = 40

NC = 2
NS = 16
NW = NC * NS
EPT = N_EDGES // NW
K = 100
NCH = EPT // K
N_PAD = 10240
ROWS_PER_TILE = N_PAD // NS
ZROWS = 80
LANES = 16


def _sc_degree(dst2):
    mesh = plsc.VectorSubcoreMesh(core_axis_name="c", subcore_axis_name="s")

    @functools.partial(
        pl.kernel,
        out_type=jax.ShapeDtypeStruct((NW, N_NODES), jnp.float32),
        mesh=mesh,
        scratch_types=[
            pltpu.VMEM((EPT,), jnp.int32),
            pltpu.VMEM((N_NODES,), jnp.float32),
        ],
        compiler_params=pltpu.CompilerParams(needs_layout_passes=False),
    )
    def deg_kernel(dst_hbm, out_hbm, idx_v, deg_v):
        c = lax.axis_index("c")
        s = lax.axis_index("s")
        wid = c * NS + s
        pltpu.sync_copy(dst_hbm.at[wid], idx_v)

        def zero_body(i, carry):
            deg_v[pl.ds(i * LANES, LANES)] = jnp.zeros((LANES,), jnp.float32)
            return carry

        lax.fori_loop(0, N_NODES // LANES, zero_body, 0)

        ones = jnp.ones((LANES,), jnp.float32)

        def acc_body(i, carry):
            idx = idx_v[pl.ds(i * LANES, LANES)]
            plsc.addupdate_scatter(deg_v, [idx], ones)
            return carry

        lax.fori_loop(0, EPT // LANES, acc_body, 0)
        pltpu.sync_copy(deg_v, out_hbm.at[wid])

    return deg_kernel(dst2)


def _sc_aggregate(y, src3, dst4):
    mesh = plsc.VectorSubcoreMesh(core_axis_name="c", subcore_axis_name="s")

    @functools.partial(
        pl.kernel,
        out_type=jax.ShapeDtypeStruct((NC, N_PAD, NHID), jnp.float32),
        mesh=mesh,
        scratch_types=[
            pltpu.VMEM((NCH, K), jnp.int32),
            pltpu.VMEM((1, K), jnp.int32),
            pltpu.VMEM((1, K), jnp.int32),
            pltpu.VMEM((K, NHID), jnp.float32),
            pltpu.VMEM((K, NHID), jnp.float32),
            pltpu.VMEM_SHARED((N_PAD, NHID), jnp.float32),
            pltpu.SemaphoreType.DMA,
            pltpu.SemaphoreType.DMA,
            pltpu.SemaphoreType.DMA,
            pltpu.SemaphoreType.DMA,
            pltpu.SemaphoreType.DMA,
        ],
    )
    def agg_kernel(y_hbm, src_hbm, dst_hbm, out_hbm,
                   src_v, db0, db1, buf0, buf1, accum,
                   gsem0, gsem1, dsem0, dsem1, zsem):
        c = lax.axis_index("c")
        s = lax.axis_index("s")
        wid = c * NS + s
        pltpu.sync_copy(src_hbm.at[wid], src_v)
        pltpu.async_copy(dst_hbm.at[wid, 0], db0, dsem0)
        pltpu.async_copy(y_hbm.at[src_v.at[0]], buf0, gsem0)

        def zb(i, carry):
            r = i // (NHID // LANES)
            q = i % (NHID // LANES)
            buf1[r, pl.ds(q * LANES, LANES)] = jnp.zeros((LANES,), jnp.float32)
            return carry

        lax.fori_loop(0, ZROWS * (NHID // LANES), zb, 0)
        base = s * ROWS_PER_TILE

        def zcopy(j, carry):
            off = pl.multiple_of(base + j * ZROWS, 8)
            pltpu.async_copy(buf1.at[pl.ds(0, ZROWS)],
                             accum.at[pl.ds(off, ZROWS)], zsem)
            return carry

        lax.fori_loop(0, ROWS_PER_TILE // ZROWS, zcopy, 0)

        def zwait(j, carry):
            pltpu.make_async_copy(buf1.at[pl.ds(0, ZROWS)],
                                  accum.at[pl.ds(base, ZROWS)], zsem).wait()
            return carry

        lax.fori_loop(0, ROWS_PER_TILE // ZROWS, zwait, 0)
        plsc.subcore_barrier()

        pltpu.async_copy(dst_hbm.at[wid, 1], db1, dsem1)
        pltpu.async_copy(y_hbm.at[src_v.at[1]], buf1, gsem1)

        def pair(i2, carry):
            i = 2 * i2
            pltpu.make_async_copy(y_hbm.at[src_v.at[i]], buf0, gsem0).wait()
            pltpu.make_async_copy(dst_hbm.at[wid, i], db0, dsem0).wait()
            pltpu.sync_copy(buf0, accum.at[db0.at[0]], add=True)

            @pl.when(i2 + 1 < NCH // 2)
            def _():
                pltpu.async_copy(dst_hbm.at[wid, i + 2], db0, dsem0)
                pltpu.async_copy(y_hbm.at[src_v.at[i + 2]], buf0, gsem0)

            pltpu.make_async_copy(y_hbm.at[src_v.at[i + 1]], buf1, gsem1).wait()
            pltpu.make_async_copy(dst_hbm.at[wid, i + 1], db1, dsem1).wait()
            pltpu.sync_copy(buf1, accum.at[db1.at[0]], add=True)

            @pl.when(i2 + 1 < NCH // 2)
            def _():
                pltpu.async_copy(dst_hbm.at[wid, i + 3], db1, dsem1)
                pltpu.async_copy(y_hbm.at[src_v.at[i + 3]], buf1, gsem1)

            return carry

        lax.fori_loop(0, NCH // 2, pair, 0)
        plsc.subcore_barrier()

        pltpu.sync_copy(accum.at[pl.ds(base, ROWS_PER_TILE)],
                        out_hbm.at[c, pl.ds(base, ROWS_PER_TILE)])

    return agg_kernel(y, src3, dst4)


_ROWS = 2000


def _tc_prep(deg_part_t, x, W1):
    def body(degp_ref, x_ref, w_ref, dinv_ref, y_ref):
        deg = jnp.sum(degp_ref[...], axis=1) + 1.0
        dinv = lax.rsqrt(deg)
        dinv_ref[...] = dinv[:, None]
        xw = jnp.dot(x_ref[...], w_ref[...], preferred_element_type=jnp.float32)
        y_ref[...] = xw * dinv[:, None]

    return pl.pallas_call(
        body,
        grid=(N_NODES // _ROWS,),
        in_specs=[
            pl.BlockSpec((_ROWS, NW), lambda i: (i, 0)),
            pl.BlockSpec((_ROWS, NFEAT), lambda i: (i, 0)),
            pl.BlockSpec((NFEAT, NHID), lambda i: (0, 0)),
        ],
        out_specs=[
            pl.BlockSpec((_ROWS, 1), lambda i: (i, 0)),
            pl.BlockSpec((_ROWS, NHID), lambda i: (i, 0)),
        ],
        out_shape=[
            jax.ShapeDtypeStruct((N_NODES, 1), jnp.float32),
            jax.ShapeDtypeStruct((N_NODES, NHID), jnp.float32),
        ],
    )(deg_part_t, x, W1)


def _tc_mid(part, y_prev, dinv, b, Wn):
    def body(p_ref, y_ref, dinv_ref, b_ref, w_ref, out_ref):
        agg = p_ref[0] + p_ref[1] + y_ref[...]
        h = jnp.maximum(agg * dinv_ref[...] + b_ref[...], 0.0)
        out_ref[...] = jnp.dot(
            h, w_ref[...], preferred_element_type=jnp.float32) * dinv_ref[...]

    return pl.pallas_call(
        body,
        grid=(N_NODES // _ROWS,),
        in_specs=[
            pl.BlockSpec((NC, _ROWS, NHID), lambda i: (0, i, 0)),
            pl.BlockSpec((_ROWS, NHID), lambda i: (i, 0)),
            pl.BlockSpec((_ROWS, 1), lambda i: (i, 0)),
            pl.BlockSpec((1, NHID), lambda i: (0, 0)),
            pl.BlockSpec((NHID, NHID), lambda i: (0, 0)),
        ],
        out_specs=pl.BlockSpec((_ROWS, NHID), lambda i: (i, 0)),
        out_shape=jax.ShapeDtypeStruct((N_NODES, NHID), jnp.float32),
    )(part, y_prev, dinv, b, Wn)


def _tc_final(part, y_prev, dinv, b2, Wl, bl):
    def body(p_ref, y_ref, dinv_ref, b_ref, wl_ref, bl_ref, out_ref):
        agg = p_ref[0] + p_ref[1] + y_ref[...]
        h = jnp.maximum(agg * dinv_ref[...] + b_ref[...], 0.0)
        logits = jnp.dot(h, wl_ref[...],
                         preferred_element_type=jnp.float32) + bl_ref[...]
        m = jnp.max(logits, axis=1, keepdims=True)
        e = jnp.exp(logits - m)
        out_ref[...] = e / jnp.sum(e, axis=1, keepdims=True)

    return pl.pallas_call(
        body,
        grid=(N_NODES // _ROWS,),
        in_specs=[
            pl.BlockSpec((NC, _ROWS, NHID), lambda i: (0, i, 0)),
            pl.BlockSpec((_ROWS, NHID), lambda i: (i, 0)),
            pl.BlockSpec((_ROWS, 1), lambda i: (i, 0)),
            pl.BlockSpec((1, NHID), lambda i: (0, 0)),
            pl.BlockSpec((NHID, NCLASS), lambda i: (0, 0)),
            pl.BlockSpec((1, NCLASS), lambda i: (0, 0)),
        ],
        out_specs=pl.BlockSpec((_ROWS, NCLASS), lambda i: (i, 0)),
        out_shape=jax.ShapeDtypeStruct((N_NODES, NCLASS), jnp.float32),
    )(part, y_prev, dinv, b2, Wl, bl)


def kernel(x, edge_index, W1, b1, W2, b2, Wl, bl):
    src = edge_index[0].astype(jnp.int32)
    dst = edge_index[1].astype(jnp.int32)
    dst2 = dst.reshape(NW, EPT)
    src3 = src.reshape(NW, NCH, K)
    dst4 = dst.reshape(NW, NCH, 1, K)

    deg_part = _sc_degree(dst2)
    dinv, y1 = _tc_prep(deg_part.T, x, W1)
    part1 = _sc_aggregate(y1, src3, dst4)
    y2 = _tc_mid(part1, y1, dinv, b1.reshape(1, NHID), W2)
    part2 = _sc_aggregate(y2, src3, dst4)
    return _tc_final(part2, y2, dinv, b2.reshape(1, NHID),
                     Wl, bl.reshape(1, NCLASS))


# speedup vs baseline: 34.4361x; 1.0322x over previous
import functools

import jax
import jax.numpy as jnp
from jax import lax
from jax.experimental import pallas as pl
from jax.experimental.pallas import tpu as pltpu
from jax.experimental.pallas import tpu_sc as plsc

N_NODES = 10000
N_EDGES = 320000
NFEAT = 128
NHID = 128
NCLASS = 40

NC = 2
NS = 16
NW = NC * NS
EPT = N_EDGES // NW
K = 125
NCH = EPT // K
N_PAD = 10240
ROWS_PER_TILE = N_PAD // NS
ZROWS = 80
LANES = 16


def _sc_degree(dst2):
    mesh = plsc.VectorSubcoreMesh(core_axis_name="c", subcore_axis_name="s")

    @functools.partial(
        pl.kernel,
        out_type=jax.ShapeDtypeStruct((NW, N_NODES), jnp.float32),
        mesh=mesh,
        scratch_types=[
            pltpu.VMEM((EPT,), jnp.int32),
            pltpu.VMEM((N_NODES,), jnp.float32),
        ],
        compiler_params=pltpu.CompilerParams(needs_layout_passes=False),
    )
    def deg_kernel(dst_hbm, out_hbm, idx_v, deg_v):
        c = lax.axis_index("c")
        s = lax.axis_index("s")
        wid = c * NS + s
        pltpu.sync_copy(dst_hbm.at[wid], idx_v)

        def zero_body(i, carry):
            deg_v[pl.ds(i * LANES, LANES)] = jnp.zeros((LANES,), jnp.float32)
            return carry

        lax.fori_loop(0, N_NODES // LANES, zero_body, 0)

        ones = jnp.ones((LANES,), jnp.float32)

        def acc_body(i, carry):
            idx = idx_v[pl.ds(i * LANES, LANES)]
            plsc.addupdate_scatter(deg_v, [idx], ones)
            return carry

        lax.fori_loop(0, EPT // LANES, acc_body, 0)
        pltpu.sync_copy(deg_v, out_hbm.at[wid])

    return deg_kernel(dst2)


def _sc_aggregate(y, src3, dst4):
    mesh = plsc.VectorSubcoreMesh(core_axis_name="c", subcore_axis_name="s")

    @functools.partial(
        pl.kernel,
        out_type=jax.ShapeDtypeStruct((NC, N_PAD, NHID), jnp.float32),
        mesh=mesh,
        scratch_types=[
            pltpu.VMEM((NCH, K), jnp.int32),
            pltpu.VMEM((1, K), jnp.int32),
            pltpu.VMEM((1, K), jnp.int32),
            pltpu.VMEM((K, NHID), jnp.float32),
            pltpu.VMEM((K, NHID), jnp.float32),
            pltpu.VMEM_SHARED((N_PAD, NHID), jnp.float32),
            pltpu.SemaphoreType.DMA,
            pltpu.SemaphoreType.DMA,
            pltpu.SemaphoreType.DMA,
            pltpu.SemaphoreType.DMA,
            pltpu.SemaphoreType.DMA,
        ],
    )
    def agg_kernel(y_hbm, src_hbm, dst_hbm, out_hbm,
                   src_v, db0, db1, buf0, buf1, accum,
                   gsem0, gsem1, dsem0, dsem1, zsem):
        c = lax.axis_index("c")
        s = lax.axis_index("s")
        wid = c * NS + s
        pltpu.sync_copy(src_hbm.at[wid], src_v)
        pltpu.async_copy(dst_hbm.at[wid, 0], db0, dsem0)
        pltpu.async_copy(y_hbm.at[src_v.at[0]], buf0, gsem0)

        def zb(i, carry):
            r = i // (NHID // LANES)
            q = i % (NHID // LANES)
            buf1[r, pl.ds(q * LANES, LANES)] = jnp.zeros((LANES,), jnp.float32)
            return carry

        lax.fori_loop(0, ZROWS * (NHID // LANES), zb, 0)
        base = s * ROWS_PER_TILE

        def zcopy(j, carry):
            off = pl.multiple_of(base + j * ZROWS, 8)
            pltpu.async_copy(buf1.at[pl.ds(0, ZROWS)],
                             accum.at[pl.ds(off, ZROWS)], zsem)
            return carry

        lax.fori_loop(0, ROWS_PER_TILE // ZROWS, zcopy, 0)

        def zwait(j, carry):
            pltpu.make_async_copy(buf1.at[pl.ds(0, ZROWS)],
                                  accum.at[pl.ds(base, ZROWS)], zsem).wait()
            return carry

        lax.fori_loop(0, ROWS_PER_TILE // ZROWS, zwait, 0)
        plsc.subcore_barrier()

        pltpu.async_copy(dst_hbm.at[wid, 1], db1, dsem1)
        pltpu.async_copy(y_hbm.at[src_v.at[1]], buf1, gsem1)

        def pair(i2, carry):
            i = 2 * i2
            pltpu.make_async_copy(y_hbm.at[src_v.at[i]], buf0, gsem0).wait()
            pltpu.make_async_copy(dst_hbm.at[wid, i], db0, dsem0).wait()
            pltpu.sync_copy(buf0, accum.at[db0.at[0]], add=True)

            @pl.when(i2 + 1 < NCH // 2)
            def _():
                pltpu.async_copy(dst_hbm.at[wid, i + 2], db0, dsem0)
                pltpu.async_copy(y_hbm.at[src_v.at[i + 2]], buf0, gsem0)

            pltpu.make_async_copy(y_hbm.at[src_v.at[i + 1]], buf1, gsem1).wait()
            pltpu.make_async_copy(dst_hbm.at[wid, i + 1], db1, dsem1).wait()
            pltpu.sync_copy(buf1, accum.at[db1.at[0]], add=True)

            @pl.when(i2 + 1 < NCH // 2)
            def _():
                pltpu.async_copy(dst_hbm.at[wid, i + 3], db1, dsem1)
                pltpu.async_copy(y_hbm.at[src_v.at[i + 3]], buf1, gsem1)

            return carry

        lax.fori_loop(0, NCH // 2, pair, 0)
        plsc.subcore_barrier()

        pltpu.sync_copy(accum.at[pl.ds(base, ROWS_PER_TILE)],
                        out_hbm.at[c, pl.ds(base, ROWS_PER_TILE)])

    return agg_kernel(y, src3, dst4)


_ROWS = 2000


def _tc_prep(deg_part_t, x, W1):
    def body(degp_ref, x_ref, w_ref, dinv_ref, y_ref):
        deg = jnp.sum(degp_ref[...], axis=1) + 1.0
        dinv = lax.rsqrt(deg)
        dinv_ref[...] = dinv[:, None]
        xw = jnp.dot(x_ref[...], w_ref[...], preferred_element_type=jnp.float32)
        y_ref[...] = xw * dinv[:, None]

    return pl.pallas_call(
        body,
        grid=(N_NODES // _ROWS,),
        in_specs=[
            pl.BlockSpec((_ROWS, NW), lambda i: (i, 0)),
            pl.BlockSpec((_ROWS, NFEAT), lambda i: (i, 0)),
            pl.BlockSpec((NFEAT, NHID), lambda i: (0, 0)),
        ],
        out_specs=[
            pl.BlockSpec((_ROWS, 1), lambda i: (i, 0)),
            pl.BlockSpec((_ROWS, NHID), lambda i: (i, 0)),
        ],
        out_shape=[
            jax.ShapeDtypeStruct((N_NODES, 1), jnp.float32),
            jax.ShapeDtypeStruct((N_NODES, NHID), jnp.float32),
        ],
    )(deg_part_t, x, W1)


def _tc_mid(part, y_prev, dinv, b, Wn):
    def body(p_ref, y_ref, dinv_ref, b_ref, w_ref, out_ref):
        agg = p_ref[0] + p_ref[1] + y_ref[...]
        h = jnp.maximum(agg * dinv_ref[...] + b_ref[...], 0.0)
        out_ref[...] = jnp.dot(
            h, w_ref[...], preferred_element_type=jnp.float32) * dinv_ref[...]

    return pl.pallas_call(
        body,
        grid=(N_NODES // _ROWS,),
        in_specs=[
            pl.BlockSpec((NC, _ROWS, NHID), lambda i: (0, i, 0)),
            pl.BlockSpec((_ROWS, NHID), lambda i: (i, 0)),
            pl.BlockSpec((_ROWS, 1), lambda i: (i, 0)),
            pl.BlockSpec((1, NHID), lambda i: (0, 0)),
            pl.BlockSpec((NHID, NHID), lambda i: (0, 0)),
        ],
        out_specs=pl.BlockSpec((_ROWS, NHID), lambda i: (i, 0)),
        out_shape=jax.ShapeDtypeStruct((N_NODES, NHID), jnp.float32),
    )(part, y_prev, dinv, b, Wn)


def _tc_final(part, y_prev, dinv, b2, Wl, bl):
    def body(p_ref, y_ref, dinv_ref, b_ref, wl_ref, bl_ref, out_ref):
        agg = p_ref[0] + p_ref[1] + y_ref[...]
        h = jnp.maximum(agg * dinv_ref[...] + b_ref[...], 0.0)
        logits = jnp.dot(h, wl_ref[...],
                         preferred_element_type=jnp.float32) + bl_ref[...]
        m = jnp.max(logits, axis=1, keepdims=True)
        e = jnp.exp(logits - m)
        out_ref[...] = e / jnp.sum(e, axis=1, keepdims=True)

    return pl.pallas_call(
        body,
        grid=(N_NODES // _ROWS,),
        in_specs=[
            pl.BlockSpec((NC, _ROWS, NHID), lambda i: (0, i, 0)),
            pl.BlockSpec((_ROWS, NHID), lambda i: (i, 0)),
            pl.BlockSpec((_ROWS, 1), lambda i: (i, 0)),
            pl.BlockSpec((1, NHID), lambda i: (0, 0)),
            pl.BlockSpec((NHID, NCLASS), lambda i: (0, 0)),
            pl.BlockSpec((1, NCLASS), lambda i: (0, 0)),
        ],
        out_specs=pl.BlockSpec((_ROWS, NCLASS), lambda i: (i, 0)),
        out_shape=jax.ShapeDtypeStruct((N_NODES, NCLASS), jnp.float32),
    )(part, y_prev, dinv, b2, Wl, bl)


def kernel(x, edge_index, W1, b1, W2, b2, Wl, bl):
    src = edge_index[0].astype(jnp.int32)
    dst = edge_index[1].astype(jnp.int32)
    dst2 = dst.reshape(NW, EPT)
    src3 = src.reshape(NW, NCH, K)
    dst4 = dst.reshape(NW, NCH, 1, K)

    deg_part = _sc_degree(dst2)
    dinv, y1 = _tc_prep(deg_part.T, x, W1)
    part1 = _sc_aggregate(y1, src3, dst4)
    y2 = _tc_mid(part1, y1, dinv, b1.reshape(1, NHID), W2)
    part2 = _sc_aggregate(y2, src3, dst4)
    return _tc_final(part2, y2, dinv, b2.reshape(1, NHID),
                     Wl, bl.reshape(1, NCLASS))
